# baseline (speedup 1.0000x reference)
import jax
import jax.numpy as jnp
from jax import lax
from jax.experimental import pallas as pl
from jax.experimental.pallas import tpu as pltpu
from jax.experimental.pallas import tpu_sc as plsc

_B = 16384
_F1 = 26
_F2 = 100
_V1 = 1000000
_V2 = 100001
_V2PAD = 100096

_NC = 2
_NS = 16
_L = 16
_NW = _NC * _NS
_SPW = _B // _NW
_SUB = 128
_NSUB = _SPW // _SUB
_JB = _SUB // _L


def _sc_body(x1f_hbm, x2_hbm, w1_hbm, w2_hbm, bias_hbm, out_hbm,
             w2_v, x1s_v, x2s_v, idx1t_v, g1_v, outs_v, bias_v, sem):
    cid = lax.axis_index("c")
    sid = lax.axis_index("s")
    wid = sid * _NC + cid
    base = wid * _SPW

    pltpu.sync_copy(w2_hbm, w2_v)
    pltpu.sync_copy(bias_hbm, bias_v)

    iota = lax.iota(jnp.int32, _L)
    rows_j = [iota + (_L * j) for j in range(_JB)]
    row1_j = [(iota + (_L * j)) * _F1 for j in range(_JB)]
    ones = jnp.ones((_L,), jnp.int32)
    bias = bias_v[...]

    def subblock(c, _):
        row0 = base + c * _SUB
        pltpu.sync_copy(x1f_hbm.at[pl.ds(row0 * _F1, _SUB * _F1)], x1s_v)
        pltpu.sync_copy(x2_hbm.at[pl.ds(row0, _SUB), :], x2s_v)

        def tr_f(f, carry):
            off = ones * f
            for j in range(_JB):
                v = plsc.load_gather(x1s_v, [row1_j[j] + off])
                idx1t_v[pl.ds(f * _SUB + _L * j, _L)] = v
            return carry
        lax.fori_loop(0, _F1, tr_f, 0)

        def fire(f, carry):
            pltpu.async_copy(w1_hbm.at[idx1t_v.at[pl.ds(f * _SUB, _SUB)]],
                             g1_v.at[pl.ds(f * _SUB, _SUB)], sem)
            return carry
        lax.fori_loop(0, _F1, fire, 0)

        def f2_loop(f, accs):
            cols = ones * f
            new = []
            for j in range(_JB):
                inds = plsc.load_gather(x2s_v, [rows_j[j], cols])
                vals = plsc.load_gather(w2_v, [inds])
                new.append(accs[j] + vals)
            return tuple(new)
        zacc = tuple(jnp.zeros((_L,), jnp.float32) for _ in range(_JB))
        accs = lax.fori_loop(0, _F2, f2_loop, zacc)

        def drain(f, carry):
            pltpu.make_async_copy(
                w1_hbm.at[idx1t_v.at[pl.ds(f * _SUB, _SUB)]],
                g1_v.at[pl.ds(f * _SUB, _SUB)], sem).wait()
            return carry
        lax.fori_loop(0, _F1, drain, 0)

        def f1_loop(f, accs_in):
            new = []
            for j in range(_JB):
                new.append(accs_in[j] + g1_v[pl.ds(f * _SUB + _L * j, _L)])
            return tuple(new)
        accs = lax.fori_loop(0, _F1, f1_loop, accs)

        for j in range(_JB):
            s = accs[j] + bias
            outs_v[pl.ds(c * _SUB + _L * j, _L)] = 1.0 / (1.0 + jnp.exp(-s))
        return 0

    lax.fori_loop(0, _NSUB, subblock, 0)
    pltpu.sync_copy(outs_v, out_hbm.at[pl.ds(base, _SPW)])


@jax.jit
def _run(x1f, x2, w1, w2, bias1):
    mesh = plsc.VectorSubcoreMesh(
        core_axis_name="c", subcore_axis_name="s",
        num_cores=_NC, num_subcores=_NS)
    f = pl.kernel(
        _sc_body,
        out_type=jax.ShapeDtypeStruct((_B,), jnp.float32),
        mesh=mesh,
        scratch_types=[
            pltpu.VMEM((_V2PAD,), jnp.float32),
            pltpu.VMEM((_SUB * _F1,), jnp.int32),
            pltpu.VMEM((_SUB, _F2), jnp.int32),
            pltpu.VMEM((_F1 * _SUB,), jnp.int32),
            pltpu.VMEM((_F1 * _SUB,), jnp.float32),
            pltpu.VMEM((_SPW,), jnp.float32),
            pltpu.VMEM((_L,), jnp.float32),
            pltpu.SemaphoreType.DMA,
        ],
        compiler_params=pltpu.CompilerParams(needs_layout_passes=False),
    )
    return f(x1f, x2, w1, w2, bias1)


def kernel(x1, x2, W1, W2, bias1):
    w2 = jnp.pad(W2.reshape(_V2), (0, _V2PAD - _V2))
    bias16 = jnp.broadcast_to(bias1.astype(jnp.float32), (_L,))
    return _run(x1.reshape(_B * _F1), x2, W1.reshape(_V1), w2, bias16)


# speedup vs baseline: 2.8478x; 2.8478x over previous
import jax
import jax.numpy as jnp
from jax import lax
from jax.experimental import pallas as pl
from jax.experimental.pallas import tpu as pltpu
from jax.experimental.pallas import tpu_sc as plsc

_B = 16384
_F1 = 26
_F2 = 100
_V1 = 1000000
_V2 = 100001
_V2PAD = 100096

_NC = 2
_NS = 16
_L = 16
_NW = _NC * _NS
_SPW = _B // _NW
_SUB = 128
_NSUB = _SPW // _SUB
_JB = _SUB // _L


def _sc_body(x1t_hbm, x2t_hbm, w1t_hbm, w2t_hbm, bias_hbm, out_hbm,
             w2_v, x1ts_v, x2ts_v, g1_v, outs_v, bias_v, idx_v,
             sem, sem2, sem3):
    w1_hbm = w1t_hbm.at[0]
    w2_hbm = w2t_hbm.at[0]
    cid = lax.axis_index("c")
    sid = lax.axis_index("s")
    wid = sid * _NC + cid
    base = wid * _SPW

    iota = lax.iota(jnp.int32, _L)
    zeros = iota * 0

    idx_v[pl.ds(0, _L)] = zeros
    idx_v[pl.ds(_L, _L)] = zeros + (_V2 - 1)
    cp_pref = pltpu.async_copy(
        w2_hbm.at[pl.ds(0, _V2 - 1)], w2_v.at[pl.ds(0, _V2 - 1)], sem2)
    cp_tail = pltpu.async_copy(
        w2_hbm.at[idx_v.at[pl.ds(_L, _L)]],
        w2_v.at[pl.ds(_V2 - 1, _L)], sem2)
    cp_bias = pltpu.async_copy(
        bias_hbm.at[idx_v.at[pl.ds(0, _L)]], bias_v, sem2)

    rows_j = [iota + (_L * j) for j in range(_JB)]
    ones = jnp.ones((_L,), jnp.int32)

    def subblock(c, _):
        row0 = base + c * _SUB
        cp_x2 = pltpu.async_copy(
            x2t_hbm.at[:, pl.ds(row0, _SUB)], x2ts_v, sem3)
        pltpu.sync_copy(x1t_hbm.at[:, pl.ds(row0, _SUB)], x1ts_v)

        def fire(f, carry):
            pltpu.async_copy(w1_hbm.at[x1ts_v.at[f]],
                             g1_v.at[pl.ds(f * _SUB, _SUB)], sem)
            return carry
        lax.fori_loop(0, _F1, fire, 0)

        @pl.when(c == 0)
        def _wait_table():
            cp_pref.wait()
            cp_tail.wait()
            cp_bias.wait()
        cp_x2.wait()

        def f2_loop(f, accs):
            new = []
            for j in range(_JB):
                inds = x2ts_v[f, pl.ds(_L * j, _L)]
                vals = plsc.load_gather(w2_v, [inds])
                new.append(accs[j] + vals)
            return tuple(new)
        zacc = tuple(jnp.zeros((_L,), jnp.float32) for _ in range(_JB))
        accs = lax.fori_loop(0, _F2, f2_loop, zacc, unroll=2)

        def drain(f, carry):
            pltpu.make_async_copy(
                w1_hbm.at[x1ts_v.at[f]],
                g1_v.at[pl.ds(f * _SUB, _SUB)], sem).wait()
            return carry
        lax.fori_loop(0, _F1, drain, 0)

        def f1_loop(f, accs_in):
            new = []
            for j in range(_JB):
                new.append(accs_in[j] + g1_v[pl.ds(f * _SUB + _L * j, _L)])
            return tuple(new)
        accs = lax.fori_loop(0, _F1, f1_loop, accs)

        bias = bias_v[...]
        for j in range(_JB):
            s = accs[j] + bias
            outs_v[pl.ds(c * _SUB + _L * j, _L)] = 1.0 / (1.0 + jnp.exp(-s))
        return 0

    lax.fori_loop(0, _NSUB, subblock, 0)
    pltpu.sync_copy(outs_v, out_hbm.at[pl.ds(base, _SPW)])


@jax.jit
def _run(x1t, x2t, w1t, w2t, bias1):
    mesh = plsc.VectorSubcoreMesh(
        core_axis_name="c", subcore_axis_name="s",
        num_cores=_NC, num_subcores=_NS)
    f = pl.kernel(
        _sc_body,
        out_type=jax.ShapeDtypeStruct((_B,), jnp.float32),
        mesh=mesh,
        scratch_types=[
            pltpu.VMEM((_V2PAD,), jnp.float32),
            pltpu.VMEM((_F1, _SUB), jnp.int32),
            pltpu.VMEM((_F2, _SUB), jnp.int32),
            pltpu.VMEM((_F1 * _SUB,), jnp.float32),
            pltpu.VMEM((_SPW,), jnp.float32),
            pltpu.VMEM((_L,), jnp.float32),
            pltpu.VMEM((2 * _L,), jnp.int32),
            pltpu.SemaphoreType.DMA,
            pltpu.SemaphoreType.DMA,
            pltpu.SemaphoreType.DMA,
        ],
        compiler_params=pltpu.CompilerParams(needs_layout_passes=False),
    )
    return f(x1t, x2t, w1t, w2t, bias1)


def kernel(x1, x2, W1, W2, bias1):
    return _run(x1.T, x2.T, W1.T, W2.T, bias1)


# speedup vs baseline: 2.9115x; 1.0224x over previous
import jax
import jax.numpy as jnp
from jax import lax
from jax.experimental import pallas as pl
from jax.experimental.pallas import tpu as pltpu
from jax.experimental.pallas import tpu_sc as plsc

_B = 16384
_F1 = 26
_F2 = 100
_V1 = 1000000
_V2 = 100001
_V2PAD = 100096

_NC = 2
_NS = 16
_L = 16
_NW = _NC * _NS
_SPW = _B // _NW
_SUB = 128
_NSUB = _SPW // _SUB
_JB = _SUB // _L


def _sc_body(x1t_hbm, x2t_hbm, w1t_hbm, w2t_hbm, bias_hbm, out_hbm,
             w2_v, x1ts_a, x1ts_b, x2ts_v, g1_a, g1_b, outs_v, bias_v,
             idx_v, sem_a, sem_b, sem2, sem3):
    w1_hbm = w1t_hbm.at[0]
    w2_hbm = w2t_hbm.at[0]
    cid = lax.axis_index("c")
    sid = lax.axis_index("s")
    wid = sid * _NC + cid
    base = wid * _SPW

    iota = lax.iota(jnp.int32, _L)
    zeros = iota * 0

    idx_v[pl.ds(0, _L)] = zeros
    idx_v[pl.ds(_L, _L)] = zeros + (_V2 - 1)
    cp_pref = pltpu.async_copy(
        w2_hbm.at[pl.ds(0, _V2 - 1)], w2_v.at[pl.ds(0, _V2 - 1)], sem2)
    cp_tail = pltpu.async_copy(
        w2_hbm.at[idx_v.at[pl.ds(_L, _L)]],
        w2_v.at[pl.ds(_V2 - 1, _L)], sem2)
    cp_bias = pltpu.async_copy(
        bias_hbm.at[idx_v.at[pl.ds(0, _L)]], bias_v, sem2)

    ones = jnp.ones((_L,), jnp.int32)
    bufs = [(x1ts_a, g1_a, sem_a), (x1ts_b, g1_b, sem_b)]

    def stage_fire(c):
        x1ts, g1, sem = bufs[c % 2]
        row0 = base + c * _SUB
        pltpu.sync_copy(x1t_hbm.at[:, pl.ds(row0, _SUB)], x1ts)

        def fire(f, carry):
            pltpu.async_copy(w1_hbm.at[x1ts.at[f]],
                             g1.at[pl.ds(f * _SUB, _SUB)], sem)
            return carry
        lax.fori_loop(0, _F1, fire, 0)

    def stage_x2(c):
        row0 = base + c * _SUB
        return pltpu.async_copy(
            x2t_hbm.at[:, pl.ds(row0, _SUB)], x2ts_v, sem3)

    cp_x2 = stage_x2(0)
    stage_fire(0)
    for c in range(_NSUB):
        if c == 0:
            cp_pref.wait()
            cp_tail.wait()
            cp_bias.wait()
        cp_x2.wait()

        def f2_loop(f, accs):
            new = []
            for j in range(_JB):
                inds = x2ts_v[f, pl.ds(_L * j, _L)]
                vals = plsc.load_gather(w2_v, [inds])
                new.append(accs[j] + vals)
            return tuple(new)
        zacc = tuple(jnp.zeros((_L,), jnp.float32) for _ in range(_JB))
        accs = lax.fori_loop(0, _F2, f2_loop, zacc, unroll=2)

        if c + 1 < _NSUB:
            cp_x2 = stage_x2(c + 1)
            stage_fire(c + 1)

        x1ts, g1, sem = bufs[c % 2]

        def drain(f, carry):
            pltpu.make_async_copy(
                w1_hbm.at[x1ts.at[f]],
                g1.at[pl.ds(f * _SUB, _SUB)], sem).wait()
            return carry
        lax.fori_loop(0, _F1, drain, 0)

        def f1_loop(f, accs_in):
            new = []
            for j in range(_JB):
                new.append(accs_in[j] + g1[pl.ds(f * _SUB + _L * j, _L)])
            return tuple(new)
        accs = lax.fori_loop(0, _F1, f1_loop, accs)

        bias = bias_v[...]
        for j in range(_JB):
            s = accs[j] + bias
            outs_v[pl.ds(c * _SUB + _L * j, _L)] = 1.0 / (1.0 + jnp.exp(-s))

    pltpu.sync_copy(outs_v, out_hbm.at[pl.ds(base, _SPW)])


@jax.jit
def _run(x1t, x2t, w1t, w2t, bias1):
    mesh = plsc.VectorSubcoreMesh(
        core_axis_name="c", subcore_axis_name="s",
        num_cores=_NC, num_subcores=_NS)
    f = pl.kernel(
        _sc_body,
        out_type=jax.ShapeDtypeStruct((_B,), jnp.float32),
        mesh=mesh,
        scratch_types=[
            pltpu.VMEM((_V2PAD,), jnp.float32),
            pltpu.VMEM((_F1, _SUB), jnp.int32),
            pltpu.VMEM((_F1, _SUB), jnp.int32),
            pltpu.VMEM((_F2, _SUB), jnp.int32),
            pltpu.VMEM((_F1 * _SUB,), jnp.float32),
            pltpu.VMEM((_F1 * _SUB,), jnp.float32),
            pltpu.VMEM((_SPW,), jnp.float32),
            pltpu.VMEM((_L,), jnp.float32),
            pltpu.VMEM((2 * _L,), jnp.int32),
            pltpu.SemaphoreType.DMA,
            pltpu.SemaphoreType.DMA,
            pltpu.SemaphoreType.DMA,
            pltpu.SemaphoreType.DMA,
        ],
        compiler_params=pltpu.CompilerParams(needs_layout_passes=False),
    )
    return f(x1t, x2t, w1t, w2t, bias1)


def kernel(x1, x2, W1, W2, bias1):
    return _run(x1.T, x2.T, W1.T, W2.T, bias1)
